# baseline (speedup 1.0000x reference)
import functools

import jax
import jax.numpy as jnp
from jax.experimental import pallas as pl
from jax.experimental.pallas import tpu as pltpu

C_PAD = 128
NK = 4


def _matmul_kernel(emb_ref, w1_ref, out_ref, acc_ref):
    k = pl.program_id(1)

    @pl.when(k == 0)
    def _init():
        acc_ref[...] = jnp.zeros_like(acc_ref)

    acc_ref[...] += jnp.dot(emb_ref[...].astype(jnp.bfloat16),
                            w1_ref[...].astype(jnp.bfloat16),
                            preferred_element_type=jnp.float32)

    @pl.when(k == NK - 1)
    def _flush():
        out_ref[0] = acc_ref[...].astype(jnp.bfloat16)


def _epilogue_kernel(tok_ref, part_ref, b1_ref, w2p_ref, p_ref, out_ref,
                     *, bs, vc):
    i = pl.program_id(0)

    wf = (part_ref[0].astype(jnp.float32) + part_ref[1].astype(jnp.float32))
    t = jnp.tanh(wf + b1_ref[...])
    u = jnp.dot(t, w2p_ref[...],
                preferred_element_type=jnp.float32)

    iota = jax.lax.broadcasted_iota(jnp.int32, (bs, vc), 1) + i * vc
    oh = (tok_ref[...] == iota).astype(jnp.bfloat16)
    counts = jnp.dot(p_ref[...], oh,
                     preferred_element_type=jnp.float32)

    out_ref[0] = jnp.dot(counts, u,
                         preferred_element_type=jnp.float32)


@jax.jit
def kernel(tokens, emb, w1, b1, w2, b2):
    B, S = tokens.shape
    V, E = emb.shape
    H = w1.shape[1]
    C = w2.shape[1]
    BS = B * S
    KC = E // (2 * NK)
    VC = V // 2

    cost1 = pl.CostEstimate(flops=2 * V * E * H, transcendentals=0,
                            bytes_accessed=4 * (V * E + E * H) + 2 * 2 * V * H)
    partials = pl.pallas_call(
        _matmul_kernel,
        out_shape=jax.ShapeDtypeStruct((2, V, H), jnp.bfloat16),
        grid=(2, NK),
        in_specs=[
            pl.BlockSpec((V, KC), lambda i, k: (0, i * NK + k)),
            pl.BlockSpec((KC, H), lambda i, k: (i * NK + k, 0)),
        ],
        out_specs=pl.BlockSpec((1, V, H), lambda i, k: (i, 0, 0)),
        scratch_shapes=[pltpu.VMEM((V, H), jnp.float32)],
        compiler_params=pltpu.CompilerParams(
            dimension_semantics=("parallel", "arbitrary")),
        cost_estimate=cost1,
    )(emb, w1)

    w2p = jnp.zeros((H, C_PAD), jnp.float32).at[:, :C].set(w2) * (1.0 / S)
    row_of = jnp.repeat(jnp.arange(B, dtype=jnp.int32), S)
    p_sel = (jnp.arange(B, dtype=jnp.int32)[:, None] == row_of[None, :]
             ).astype(jnp.bfloat16)
    tok_flat = tokens.reshape(BS, 1).astype(jnp.int32)

    cost2 = pl.CostEstimate(flops=2 * V * H * C_PAD + 2 * B * BS * V
                            + 2 * B * V * C_PAD,
                            transcendentals=V * H,
                            bytes_accessed=2 * 2 * V * H + 4 * BS)
    parts = pl.pallas_call(
        functools.partial(_epilogue_kernel, bs=BS, vc=VC),
        out_shape=jax.ShapeDtypeStruct((2, B, C_PAD), jnp.float32),
        grid=(2,),
        in_specs=[
            pl.BlockSpec((BS, 1), lambda i: (0, 0)),
            pl.BlockSpec((2, VC, H), lambda i: (0, i, 0)),
            pl.BlockSpec((1, H), lambda i: (0, 0)),
            pl.BlockSpec((H, C_PAD), lambda i: (0, 0)),
            pl.BlockSpec((B, BS), lambda i: (0, 0)),
        ],
        out_specs=pl.BlockSpec((1, B, C_PAD), lambda i: (i, 0, 0)),
        compiler_params=pltpu.CompilerParams(
            dimension_semantics=("parallel",)),
        cost_estimate=cost2,
    )(tok_flat, partials, b1, w2p, p_sel)

    return parts.sum(axis=0)[:, :C] + b2


# speedup vs baseline: 1.2332x; 1.2332x over previous
import functools

import jax
import jax.numpy as jnp
from jax.experimental import pallas as pl
from jax.experimental.pallas import tpu as pltpu

C_PAD = 128
NCH = 4


def _table_kernel(tok_ref, emb_ref, w1_ref, b1_ref, w2p_ref, p_ref, out_ref,
                  *, bs, vc):
    i = pl.program_id(0)

    embc = emb_ref[...].astype(jnp.bfloat16)
    w1c = w1_ref[...].astype(jnp.bfloat16)
    wf = jnp.dot(embc, w1c, preferred_element_type=jnp.float32)
    t = jnp.tanh(wf + b1_ref[...])
    u = jnp.dot(t, w2p_ref[...],
                preferred_element_type=jnp.float32)

    iota = jax.lax.broadcasted_iota(jnp.int32, (bs, vc), 1) + i * vc
    oh = (tok_ref[...] == iota).astype(jnp.bfloat16)
    counts = jnp.dot(p_ref[...], oh,
                     preferred_element_type=jnp.float32)

    out_ref[0] = jnp.dot(counts, u,
                         preferred_element_type=jnp.float32)


@jax.jit
def kernel(tokens, emb, w1, b1, w2, b2):
    B, S = tokens.shape
    V, E = emb.shape
    H = w1.shape[1]
    C = w2.shape[1]
    VC = V // NCH
    BS = B * S

    w2p = jnp.zeros((H, C_PAD), jnp.float32).at[:, :C].set(w2) * (1.0 / S)
    row_of = jnp.repeat(jnp.arange(B, dtype=jnp.int32), S)
    p_sel = (jnp.arange(B, dtype=jnp.int32)[:, None] == row_of[None, :]
             ).astype(jnp.bfloat16)
    tok_flat = tokens.reshape(BS, 1).astype(jnp.int32)

    flops = 2 * V * E * H + 2 * B * BS * V + 2 * B * V * C_PAD
    cost = pl.CostEstimate(flops=flops, transcendentals=V * H,
                           bytes_accessed=4 * (V * E + E * H + V * H))

    parts = pl.pallas_call(
        functools.partial(_table_kernel, bs=BS, vc=VC),
        out_shape=jax.ShapeDtypeStruct((NCH, B, C_PAD), jnp.float32),
        grid=(NCH,),
        in_specs=[
            pl.BlockSpec((BS, 1), lambda i: (0, 0)),
            pl.BlockSpec((VC, E), lambda i: (i, 0)),
            pl.BlockSpec((E, H), lambda i: (0, 0)),
            pl.BlockSpec((1, H), lambda i: (0, 0)),
            pl.BlockSpec((H, C_PAD), lambda i: (0, 0)),
            pl.BlockSpec((B, BS), lambda i: (0, 0)),
        ],
        out_specs=pl.BlockSpec((1, B, C_PAD), lambda i: (i, 0, 0)),
        compiler_params=pltpu.CompilerParams(
            dimension_semantics=("parallel",)),
        cost_estimate=cost,
    )(tok_flat, emb, w1, b1, w2p, p_sel)

    return parts.sum(axis=0)[:, :C] + b2
